# baseline (speedup 1.0000x reference)
import functools

import jax
import jax.numpy as jnp
from jax import lax
from jax.experimental import pallas as pl
from jax.experimental.pallas import tpu as pltpu
from jax.experimental.pallas import tpu_sc as plsc

_D = 64
_BATCH = 16384
_HIST = 50
_N = _BATCH * _HIST
_NC = 2
_NS = 16
_NW = _NC * _NS
_IPW = _N // _NW
_CHUNK = 128
_N_CHUNKS = _IPW // _CHUNK
_NBUF = 8
_GD = 7

_mesh = plsc.VectorSubcoreMesh(core_axis_name="c", subcore_axis_name="s")


@functools.partial(
    pl.kernel,
    out_type=jax.ShapeDtypeStruct((_N, _D), jnp.float32),
    mesh=_mesh,
    scratch_types=[
        pltpu.VMEM((_IPW,), jnp.int32),
        pltpu.VMEM((_NBUF, _CHUNK, _D), jnp.float32),
        pltpu.SemaphoreType.DMA,
        pltpu.SemaphoreType.DMA((_NBUF,)),
        pltpu.SemaphoreType.DMA((_NBUF,)),
    ],
    compiler_params=pltpu.CompilerParams(use_tc_tiling_on_sc=False),
)
def _embed_gather(idx_hbm, table_hbm, out_hbm, idx_v, rows_v,
                  idx_sem, gat_sem, out_sem):
    wid = lax.axis_index("s") * _NC + lax.axis_index("c")
    base = wid * _IPW

    pltpu.async_copy(idx_hbm.at[pl.ds(base, _IPW)], idx_v, idx_sem)
    pltpu.make_async_copy(idx_hbm.at[pl.ds(base, _IPW)], idx_v,
                          idx_sem).wait()

    def fire_gather(i, b):
        pltpu.async_copy(table_hbm.at[idx_v.at[pl.ds(i * _CHUNK, _CHUNK)]],
                         rows_v.at[b], gat_sem.at[b])

    def wait_gather(b):
        pltpu.make_async_copy(table_hbm.at[idx_v.at[pl.ds(0, _CHUNK)]],
                              rows_v.at[b], gat_sem.at[b]).wait()

    def fire_out(i, b):
        pltpu.async_copy(rows_v.at[b],
                         out_hbm.at[pl.ds(base + i * _CHUNK, _CHUNK)],
                         out_sem.at[b])

    def wait_out(b):
        pltpu.make_async_copy(rows_v.at[b],
                              out_hbm.at[pl.ds(base, _CHUNK)],
                              out_sem.at[b]).wait()

    for i in range(_GD):
        fire_gather(i, i)

    @pl.loop(_GD, _N_CHUNKS)
    def _steady(i):
        b = lax.rem(i, _NBUF)

        @pl.when(i >= _NBUF)
        def _():
            wait_out(b)

        fire_gather(i, b)

        j = i - _GD
        bj = lax.rem(j, _NBUF)
        wait_gather(bj)
        fire_out(j, bj)

    for k in range(_GD):
        j = _N_CHUNKS - _GD + k
        wait_gather(j % _NBUF)
        fire_out(j, j % _NBUF)
    for b in range(min(_NBUF, _N_CHUNKS)):
        wait_out(b)


def kernel(token_ids, embedding_mat):
    flat_idx = token_ids.astype(jnp.int32).reshape(_N)
    out = _embed_gather(flat_idx, embedding_mat)
    return out.reshape(_BATCH, _HIST, _D)


# speedup vs baseline: 1.0015x; 1.0015x over previous
import functools

import jax
import jax.numpy as jnp
from jax import lax
from jax.experimental import pallas as pl
from jax.experimental.pallas import tpu as pltpu
from jax.experimental.pallas import tpu_sc as plsc

_D = 64
_BATCH = 16384
_HIST = 50
_N = _BATCH * _HIST
_NC = 2
_NS = 16
_NW = _NC * _NS
_IPW = _N // _NW
_CHUNK = 128
_N_CHUNKS = _IPW // _CHUNK
_NBUF = 8
_GD = 6

_mesh = plsc.VectorSubcoreMesh(core_axis_name="c", subcore_axis_name="s")


@functools.partial(
    pl.kernel,
    out_type=jax.ShapeDtypeStruct((_N, _D), jnp.float32),
    mesh=_mesh,
    scratch_types=[
        pltpu.VMEM((_IPW,), jnp.int32),
        pltpu.VMEM((_NBUF, _CHUNK, _D), jnp.float32),
        pltpu.SemaphoreType.DMA,
        pltpu.SemaphoreType.DMA((_NBUF,)),
        pltpu.SemaphoreType.DMA((_NBUF,)),
    ],
    compiler_params=pltpu.CompilerParams(use_tc_tiling_on_sc=False),
)
def _embed_gather(idx_hbm, table_hbm, out_hbm, idx_v, rows_v,
                  idx_sem, gat_sem, out_sem):
    wid = lax.axis_index("s") * _NC + lax.axis_index("c")
    base = wid * _IPW

    pltpu.async_copy(idx_hbm.at[pl.ds(base, _IPW)], idx_v, idx_sem)
    pltpu.make_async_copy(idx_hbm.at[pl.ds(base, _IPW)], idx_v,
                          idx_sem).wait()

    def fire_gather(i, b):
        pltpu.async_copy(table_hbm.at[idx_v.at[pl.ds(i * _CHUNK, _CHUNK)]],
                         rows_v.at[b], gat_sem.at[b])

    def wait_gather(b):
        pltpu.make_async_copy(table_hbm.at[idx_v.at[pl.ds(0, _CHUNK)]],
                              rows_v.at[b], gat_sem.at[b]).wait()

    def fire_out(i, b):
        pltpu.async_copy(rows_v.at[b],
                         out_hbm.at[pl.ds(base + i * _CHUNK, _CHUNK)],
                         out_sem.at[b])

    def wait_out(b):
        pltpu.make_async_copy(rows_v.at[b],
                              out_hbm.at[pl.ds(base, _CHUNK)],
                              out_sem.at[b]).wait()

    for i in range(_GD):
        fire_gather(i, i)

    @pl.loop(_GD, _N_CHUNKS)
    def _steady(i):
        b = lax.rem(i, _NBUF)

        @pl.when(i >= _NBUF)
        def _():
            wait_out(b)

        fire_gather(i, b)

        j = i - _GD
        bj = lax.rem(j, _NBUF)
        wait_gather(bj)
        fire_out(j, bj)

    for k in range(_GD):
        j = _N_CHUNKS - _GD + k
        wait_gather(j % _NBUF)
        fire_out(j, j % _NBUF)
    for b in range(min(_NBUF, _N_CHUNKS)):
        wait_out(b)


def kernel(token_ids, embedding_mat):
    flat_idx = token_ids.astype(jnp.int32).reshape(_N)
    out = _embed_gather(flat_idx, embedding_mat)
    return out.reshape(_BATCH, _HIST, _D)
